# baseline (speedup 1.0000x reference)
import functools

import jax
import jax.numpy as jnp
from jax import lax
from jax.experimental import pallas as pl
from jax.experimental.pallas import tpu as pltpu
from jax.experimental.pallas import tpu_sc as plsc

N_NODES = 10000
EMBED = 128
BT = 24

NC = 2
NS = 16
NW = NC * NS

CHUNK_A = 64
NCH_A = 1
N_A = NW * NCH_A * CHUNK_A
B_START = 2000
CHUNK_B = 88
NCH_B = 3
N_B = NW * NCH_B * CHUNK_B
DEG_LEN = B_START + N_B


def _make_sc_body(nchunks, chunk):
    def body(zin_hbm, zout_hbm, din_hbm, dout_hbm, out_hbm,
             idx_in_v, idx_out_v, rows_in_v, rows_out_v, sem):
        wid = lax.axis_index("s") * NC + lax.axis_index("c")
        rows_per_w = nchunks * chunk
        base = wid * rows_per_w
        cps = []
        for j in range(nchunks):
            off = base + j * chunk
            cps.append(pltpu.async_copy(din_hbm.at[pl.ds(off, chunk)],
                                        idx_in_v.at[j], sem))
            cps.append(pltpu.async_copy(dout_hbm.at[pl.ds(off, chunk)],
                                        idx_out_v.at[j], sem))
        for cp in cps:
            cp.wait()
        cps = []
        for j in range(nchunks):
            sl = pl.ds(j * chunk, chunk)
            cps.append(pltpu.async_copy(zin_hbm.at[idx_in_v.at[j]],
                                        rows_in_v.at[sl], sem))
            cps.append(pltpu.async_copy(zout_hbm.at[idx_out_v.at[j]],
                                        rows_out_v.at[sl], sem))
        for cp in cps:
            cp.wait()
        cps = [pltpu.async_copy(rows_in_v,
                                out_hbm.at[0, pl.ds(base, rows_per_w)], sem),
               pltpu.async_copy(rows_out_v,
                                out_hbm.at[1, pl.ds(base, rows_per_w)], sem)]
        for cp in cps:
            cp.wait()
    return body


def _make_sc(nchunks, chunk, n_total):
    return functools.partial(
        pl.kernel,
        out_type=jax.ShapeDtypeStruct((2, n_total, EMBED), jnp.float32),
        mesh=plsc.VectorSubcoreMesh(core_axis_name="c", subcore_axis_name="s"),
        scratch_types=[
            pltpu.VMEM((nchunks, chunk), jnp.int32),
            pltpu.VMEM((nchunks, chunk), jnp.int32),
            pltpu.VMEM((nchunks * chunk, EMBED), jnp.float32),
            pltpu.VMEM((nchunks * chunk, EMBED), jnp.float32),
            pltpu.SemaphoreType.DMA,
        ],
    )(_make_sc_body(nchunks, chunk))


_sc_gather_a = _make_sc(NCH_A, CHUNK_A, N_A)
_sc_gather_b = _make_sc(NCH_B, CHUNK_B, N_B)

BN = 2000
BBT = 12


def _add_body(x_ref, c_ref, o_ref):
    o_ref[...] = x_ref[...] + (c_ref[0] + c_ref[1])[None]


def _add_body_alias(prev_ref, x_ref, c_ref, o_ref):
    del prev_ref
    o_ref[...] = x_ref[...] + (c_ref[0] + c_ref[1])[None]


def _tc_add_head(xr, cent_a):
    return pl.pallas_call(
        _add_body,
        grid=(1, BT // BBT),
        in_specs=[
            pl.BlockSpec((BBT, BN, EMBED), lambda n, bt: (bt, n, 0)),
            pl.BlockSpec((2, BN, EMBED), lambda n, bt: (0, 0, 0)),
        ],
        out_specs=pl.BlockSpec((BBT, BN, EMBED), lambda n, bt: (bt, n, 0)),
        out_shape=jax.ShapeDtypeStruct((BT, N_NODES, EMBED), jnp.float32),
    )(xr, cent_a)


def _tc_add_tail(out0, xr, cent_b):
    return pl.pallas_call(
        _add_body_alias,
        grid=((N_NODES - B_START) // BN, BT // BBT),
        in_specs=[
            pl.BlockSpec(memory_space=pl.ANY),
            pl.BlockSpec((BBT, BN, EMBED), lambda n, bt: (bt, n + 1, 0)),
            pl.BlockSpec((2, BN, EMBED), lambda n, bt: (0, n, 0)),
        ],
        out_specs=pl.BlockSpec((BBT, BN, EMBED), lambda n, bt: (bt, n + 1, 0)),
        out_shape=jax.ShapeDtypeStruct((BT, N_NODES, EMBED), jnp.float32),
        input_output_aliases={0: 0},
    )(out0, xr, cent_b)


def kernel(x, z_in, z_out, in_degree, out_degree):
    din = jnp.pad(in_degree.astype(jnp.int32), (0, DEG_LEN - N_NODES))
    dout = jnp.pad(out_degree.astype(jnp.int32), (0, DEG_LEN - N_NODES))
    din_b = lax.slice(din, (B_START,), (DEG_LEN,))
    dout_b = lax.slice(dout, (B_START,), (DEG_LEN,))
    cent_a = _sc_gather_a(z_in, z_out, din, dout)
    cent_b = _sc_gather_b(z_in, z_out, din_b, dout_b)
    xr = x.reshape(BT, N_NODES, EMBED)
    out0 = _tc_add_head(xr, cent_a)
    out = _tc_add_tail(out0, xr, cent_b)
    return out.reshape(x.shape)
